# Initial kernel scaffold; baseline (speedup 1.0000x reference)
import jax
import jax.numpy as jnp
from jax.experimental import pallas as pl


def kernel(x, W_enc, b_enc, W_dec, b_dec):
    raise NotImplementedError("write your pallas kernel here")


# speedup vs baseline: 1.8069x; 1.8069x over previous
import functools

import jax
import jax.numpy as jnp
from jax import lax
from jax.experimental import pallas as pl
from jax.experimental.pallas import tpu as pltpu
from jax.experimental.pallas import tpu_sc as plsc

DM = 4096
NF = 32768
BT = 32
KTOP = 32
L = 16
NC, NS = 2, 16
NV = NF // L
GRP = 4

BN = 512


def _enc_body(x_ref, bdec_ref, w_ref, benc_ref, o_ref):
    xm = x_ref[...] - bdec_ref[...]
    o_ref[...] = (
        jnp.dot(xm, w_ref[...], preferred_element_type=jnp.float32)
        + benc_ref[...]
    )


def _encode(x, W_enc, b_enc, b_dec):
    return pl.pallas_call(
        _enc_body,
        grid=(NF // BN,),
        in_specs=[
            pl.BlockSpec((BT, DM), lambda i: (0, 0)),
            pl.BlockSpec((1, DM), lambda i: (0, 0)),
            pl.BlockSpec((DM, BN), lambda i: (0, i)),
            pl.BlockSpec((1, BN), lambda i: (0, i)),
        ],
        out_specs=pl.BlockSpec((BT, BN), lambda i: (0, i)),
        out_shape=jax.ShapeDtypeStruct((BT, NF), jnp.float32),
    )(x, b_dec.reshape(1, DM), W_enc, b_enc.reshape(1, NF))


def _merge16(hik, hii, lok, loi, sk, si):
    rk = lax.rev(sk, (0,))
    ri = lax.rev(si, (0,))
    p = lok >= rk
    ak = jnp.where(p, lok, rk)
    ai = jnp.where(p, loi, ri)
    ak, ai = plsc.sort_key_val(ak, ai, descending=True)
    rk = lax.rev(ak, (0,))
    ri = lax.rev(ai, (0,))
    p = hik >= rk
    nk = jnp.where(p, hik, rk)
    ni = jnp.where(p, hii, ri)
    mk = jnp.where(p, rk, hik)
    mi = jnp.where(p, ri, hii)
    nk, ni = plsc.sort_key_val(nk, ni, descending=True)
    mk, mi = plsc.sort_key_val(mk, mi, descending=True)
    return nk, ni, mk, mi


def _sc_body(pre_hbm, wdec_hbm, bdec_hbm, f_hbm, recon_hbm,
             row_v, buf_v, acc_v, bdec_v, idxh_v, idxl_v, sem):
    wid = lax.axis_index("s") * NC + lax.axis_index("c")
    pltpu.sync_copy(pre_hbm.at[wid], row_v)

    neg = jnp.float32(-3.0e38)
    lane = lax.iota(jnp.int32, L)

    def scan_group(g, carry):
        hik, hii, lok, loi, thr = carry
        base = g * (GRP * L)
        v0 = row_v[pl.ds(base, L)]
        v1 = row_v[pl.ds(base + L, L)]
        v2 = row_v[pl.ds(base + 2 * L, L)]
        v3 = row_v[pl.ds(base + 3 * L, L)]
        gmax = lax.reduce_max(
            jnp.maximum(jnp.maximum(v0, v1), jnp.maximum(v2, v3)), (0,))

        def hit(c):
            def one(c, v, off):
                hik, hii, lok, loi, thr = c
                vmax = lax.reduce_max(v, (0,))

                def do(c):
                    hik, hii, lok, loi, _ = c
                    sk, si = plsc.sort_key_val(v, lane + off,
                                               descending=True)
                    hik, hii, lok, loi = _merge16(hik, hii, lok, loi, sk, si)
                    return hik, hii, lok, loi, lax.reduce_min(lok, (0,))

                return lax.cond(vmax > c[4], do, lambda c: c, c)

            c = one(c, v0, base)
            c = one(c, v1, base + L)
            c = one(c, v2, base + 2 * L)
            c = one(c, v3, base + 3 * L)
            return c

        return lax.cond(gmax > thr, hit, lambda c: c, carry)

    init = (jnp.full((L,), neg), jnp.zeros((L,), jnp.int32),
            jnp.full((L,), neg), jnp.zeros((L,), jnp.int32),
            neg)
    hik, hii, lok, loi, _ = lax.fori_loop(0, NV // GRP, scan_group, init)

    zero = jnp.zeros((L,), jnp.float32)

    def zbody(i, _):
        base = i * (8 * L)
        for u in range(8):
            row_v[pl.ds(base + u * L, L)] = zero
        return 0

    lax.fori_loop(0, NV // 8, zbody, 0)
    plsc.store_scatter(row_v, [hii], jnp.maximum(hik, 0.0))
    plsc.store_scatter(row_v, [loi], jnp.maximum(lok, 0.0))
    pltpu.sync_copy(row_v, f_hbm.at[wid])

    idxh_v[...] = hii
    idxl_v[...] = loi
    pltpu.sync_copy(bdec_hbm, bdec_v)

    def accum(idx_ref, vals, first):
        pltpu.async_copy(wdec_hbm.at[idx_ref], buf_v, sem).wait()
        ws = [vals[r] for r in range(L)]
        src = bdec_v if first else acc_v

        def jbody(j, _):
            o = j * L
            a = src[pl.ds(o, L)]
            for r in range(L):
                a = a + ws[r] * buf_v[r, pl.ds(o, L)]
            acc_v[pl.ds(o, L)] = a
            return 0

        lax.fori_loop(0, DM // L, jbody, 0)

    accum(idxh_v, jnp.maximum(hik, 0.0), True)
    accum(idxl_v, jnp.maximum(lok, 0.0), False)
    pltpu.sync_copy(acc_v, recon_hbm.at[wid])


def _decode_topk(pre, W_dec, b_dec):
    mesh = plsc.VectorSubcoreMesh(
        core_axis_name="c", subcore_axis_name="s",
        num_cores=NC, num_subcores=NS)
    fn = functools.partial(
        pl.kernel,
        out_type=(jax.ShapeDtypeStruct((BT, NF), jnp.float32),
                  jax.ShapeDtypeStruct((BT, DM), jnp.float32)),
        mesh=mesh,
        scratch_types=[
            pltpu.VMEM((NF,), jnp.float32),
            pltpu.VMEM((L, DM), jnp.float32),
            pltpu.VMEM((DM,), jnp.float32),
            pltpu.VMEM((DM,), jnp.float32),
            pltpu.VMEM((L,), jnp.int32),
            pltpu.VMEM((L,), jnp.int32),
            pltpu.SemaphoreType.DMA,
        ],
        compiler_params=pltpu.CompilerParams(needs_layout_passes=False),
    )(_sc_body)
    return fn(pre, W_dec, b_dec)


def kernel(x, W_enc, b_enc, W_dec, b_dec):
    pre = _encode(x, W_enc, b_enc, b_dec)
    f, recon = _decode_topk(pre, W_dec, b_dec)
    return (recon, f)
